# baseline (speedup 1.0000x reference)
import jax
import jax.numpy as jnp
from jax import lax
from jax.experimental import pallas as pl
from jax.experimental.pallas import tpu as pltpu
from jax.experimental.pallas import tpu_sc as plsc

B = 1024
S = 200
D = 128
NR = 4
BB = 16

NC = 2
NS = 16
NW = NC * NS
N_ROWS = B * S
RPW = N_ROWS // NW
CR = 32
NCHUNK = RPW // CR


def _ftab_body(rt_ref, pos_ref, b_ref, f_ref):
    f_ref[...] = (rt_ref[...][:, None, :] + pos_ref[...][None, :, :]
                  + b_ref[...][None])


def _enc_body(inp_ref, pos_ref, enc_ref):
    enc_ref[...] = inp_ref[...] + pos_ref[...][None]


def _dec_body(f_hbm, r_hbm, e_hbm, w_hbm, dec_hbm,
              f_v, r_v, e_v, w_v, out_v, sem0, sem1):
    wid = lax.axis_index("s") * NC + lax.axis_index("c")
    base = wid * RPW
    pltpu.sync_copy(f_hbm, f_v)
    pltpu.sync_copy(r_hbm.at[pl.ds(base, RPW)], r_v)
    pltpu.sync_copy(e_hbm.at[pl.ds(base, RPW)], e_v)
    pltpu.sync_copy(w_hbm, w_v)
    wregs = [w_v[pl.ds(c * 16, 16)] for c in range(8)]
    sems = (sem0, sem1)

    def fill_and_send(g, slot):
        cbase = g * CR
        out = out_v.at[slot]

        def grp(gi, carry):
            gbase = gi * 16
            r16 = r_v[pl.ds(cbase + gbase, 16)]
            e16 = e_v[pl.ds(cbase + gbase, 16)]
            for j in range(16):
                rs = r16[j]
                es = e16[j]
                s = lax.rem(base + cbase + gbase + j, S)
                for c in range(8):
                    fc = f_v[rs, s, pl.ds(c * 16, 16)]
                    out[gbase + j, pl.ds(c * 16, 16)] = fc + es * wregs[c]
            return carry

        lax.fori_loop(0, CR // 16, grp, 0, unroll=False)
        pltpu.make_async_copy(
            out, dec_hbm.at[pl.ds(base + cbase, CR)], sems[slot]).start()

    def wait_slot(slot):
        pltpu.make_async_copy(
            out_v.at[slot], dec_hbm.at[pl.ds(0, CR)], sems[slot]).wait()

    fill_and_send(0, 0)
    fill_and_send(1, 1)

    def super_step(gg, carry):
        for b in range(2):
            wait_slot(b)
            fill_and_send(2 * gg + b, b)
        return carry

    lax.fori_loop(1, NCHUNK // 2, super_step, 0, unroll=False)
    wait_slot(0)
    wait_slot(1)


def kernel(input_nlp_embedding, input_r, in_elapsed_time, output_nlp_embedding,
           response_table, et_W, et_b, position_table):
    b2 = et_b.reshape(1, D)
    ftab = pl.pallas_call(
        _ftab_body,
        in_specs=[
            pl.BlockSpec((NR, D), lambda: (0, 0)),
            pl.BlockSpec((S, D), lambda: (0, 0)),
            pl.BlockSpec((1, D), lambda: (0, 0)),
        ],
        out_specs=pl.BlockSpec((NR, S, D), lambda: (0, 0, 0)),
        out_shape=jax.ShapeDtypeStruct((NR, S, D), jnp.float32),
    )(response_table, position_table, b2)

    mesh = plsc.VectorSubcoreMesh(core_axis_name="c", subcore_axis_name="s")
    dec_flat = pl.kernel(
        _dec_body,
        out_type=jax.ShapeDtypeStruct((N_ROWS, D), jnp.float32),
        mesh=mesh,
        scratch_types=[
            pltpu.VMEM((NR, S, D), jnp.float32),
            pltpu.VMEM((RPW,), jnp.int32),
            pltpu.VMEM((RPW,), jnp.float32),
            pltpu.VMEM((D,), jnp.float32),
            pltpu.VMEM((2, CR, D), jnp.float32),
            pltpu.SemaphoreType.DMA,
            pltpu.SemaphoreType.DMA,
        ],
        compiler_params=pltpu.CompilerParams(use_tc_tiling_on_sc=False),
    )(ftab, input_r.reshape(N_ROWS), in_elapsed_time.reshape(N_ROWS),
      et_W.reshape(D))
    dec = dec_flat.reshape(B, S, D)

    enc = pl.pallas_call(
        _enc_body,
        grid=(B // BB,),
        in_specs=[
            pl.BlockSpec((BB, S, D), lambda i: (i, 0, 0)),
            pl.BlockSpec((S, D), lambda i: (0, 0)),
        ],
        out_specs=pl.BlockSpec((BB, S, D), lambda i: (i, 0, 0)),
        out_shape=jax.ShapeDtypeStruct((B, S, D), jnp.float32),
    )(input_nlp_embedding, position_table)

    return (enc, dec, output_nlp_embedding)
